# speedup vs baseline: 1.2190x; 1.2190x over previous
import functools

import jax
import jax.numpy as jnp
from jax import lax
from jax.experimental import pallas as pl
from jax.experimental.pallas import tpu as pltpu
from jax.experimental.pallas import tpu_sc as plsc

_NC = 2
_NS = 16
_NW = _NC * _NS


def _sc_gather(table, idx):
    V, D = table.shape
    N = idx.shape[0]
    n_per_w = N // _NW
    R = 64
    n_chunks = n_per_w // R
    mesh = plsc.VectorSubcoreMesh(core_axis_name="c", subcore_axis_name="s")

    @functools.partial(
        pl.kernel, mesh=mesh,
        out_type=jax.ShapeDtypeStruct((N, D), jnp.float32),
        scratch_types=[
            pltpu.VMEM((n_per_w,), jnp.int32),
            pltpu.VMEM((R, D), jnp.float32),
            pltpu.SemaphoreType.DMA,
        ],
    )
    def k(table_hbm, idx_hbm, out_hbm, idx_v, rows_v, sem):
        wid = lax.axis_index("s") * _NC + lax.axis_index("c")
        base = wid * n_per_w
        pltpu.sync_copy(idx_hbm.at[pl.ds(base, n_per_w)], idx_v)

        @pl.loop(0, n_chunks)
        def _(c):
            off = c * R
            pltpu.async_copy(
                table_hbm.at[idx_v.at[pl.ds(off, R)]], rows_v, sem
            ).wait()
            pltpu.sync_copy(rows_v, out_hbm.at[pl.ds(base + off, R)])

    return k(table, idx)


def _add_body(x_ref, pe_ref, o_ref):
    o_ref[...] = x_ref[...] + pe_ref[...]


def _tc_add(x2d, pe):
    N, D = x2d.shape
    BLK = 1024
    return pl.pallas_call(
        _add_body,
        grid=(N // BLK,),
        in_specs=[pl.BlockSpec((BLK, D), lambda i: (i, 0)),
                  pl.BlockSpec((BLK, D), lambda i: (i, 0))],
        out_specs=pl.BlockSpec((BLK, D), lambda i: (i, 0)),
        out_shape=jax.ShapeDtypeStruct((N, D), jnp.float32),
    )(x2d, pe)


def kernel(x, position_indices, pe_table):
    B, S, D = x.shape
    idx = position_indices.reshape(-1).astype(jnp.int32)
    pe = _sc_gather(pe_table, idx)
    out2d = _tc_add(x.reshape(B * S, D), pe)
    return out2d.reshape(B, S, D)


# speedup vs baseline: 1.7704x; 1.4523x over previous
import functools

import jax
import jax.numpy as jnp
from jax import lax
from jax.experimental import pallas as pl
from jax.experimental.pallas import tpu as pltpu
from jax.experimental.pallas import tpu_sc as plsc

_NC = 2
_NS = 16
_NW = _NC * _NS


def _sc_gather_add(x2d, idx, table):
    V, D = table.shape
    N = idx.shape[0]
    n_per_w = N // _NW
    R = 16
    n_chunks = n_per_w // R
    mesh = plsc.VectorSubcoreMesh(core_axis_name="c", subcore_axis_name="s")

    @functools.partial(
        pl.kernel, mesh=mesh,
        out_type=jax.ShapeDtypeStruct((N, D), jnp.float32),
        scratch_types=[
            pltpu.VMEM((n_per_w,), jnp.int32),
            pltpu.VMEM((2, R, D), jnp.float32),
            pltpu.VMEM((2, R, D), jnp.float32),
            pltpu.VMEM((2, R, D), jnp.float32),
            pltpu.SemaphoreType.DMA,
            pltpu.SemaphoreType.DMA,
            pltpu.SemaphoreType.DMA,
            pltpu.SemaphoreType.DMA,
            pltpu.SemaphoreType.DMA,
            pltpu.SemaphoreType.DMA,
        ],
    )
    def k(x_hbm, idx_hbm, table_hbm, out_hbm, idx_v, pe_v, x_v, o_v,
          gs0, gs1, xs0, xs1, os0, os1):
        gsem = (gs0, gs1)
        xsem = (xs0, xs1)
        osem = (os0, os1)
        wid = lax.axis_index("s") * _NC + lax.axis_index("c")
        base = wid * n_per_w
        pltpu.sync_copy(idx_hbm.at[pl.ds(base, n_per_w)], idx_v)

        def start_fetch(c, b):
            pltpu.async_copy(
                table_hbm.at[idx_v.at[pl.ds(c * R, R)]], pe_v.at[b], gsem[b])
            pltpu.async_copy(
                x_hbm.at[pl.ds(base + c * R, R)], x_v.at[b], xsem[b])

        def wait_fetch(b):
            pltpu.make_async_copy(
                table_hbm.at[pl.ds(0, R)], pe_v.at[b], gsem[b]).wait()
            pltpu.make_async_copy(
                x_hbm.at[pl.ds(0, R)], x_v.at[b], xsem[b]).wait()

        def wait_out(b):
            pltpu.make_async_copy(
                o_v.at[b], out_hbm.at[pl.ds(0, R)], osem[b]).wait()

        start_fetch(0, 0)
        start_fetch(1, 1)

        @pl.loop(0, n_chunks, step=2)
        def _(c):
            for b in range(2):
                cc = c + b
                wait_fetch(b)

                @pl.when(cc >= 2)
                def _():
                    wait_out(b)

                @pl.loop(0, R)
                def _(r):
                    @pl.loop(0, D, step=64)
                    def _(col):
                        for u in range(4):
                            s = pl.ds(col + u * 16, 16)
                            o_v.at[b, r, s][...] = (
                                pe_v.at[b, r, s][...] + x_v.at[b, r, s][...])

                pltpu.async_copy(
                    o_v.at[b], out_hbm.at[pl.ds(base + cc * R, R)], osem[b])

                @pl.when(cc + 2 < n_chunks)
                def _():
                    start_fetch(cc + 2, b)

        wait_out(0)
        wait_out(1)

    return k(x2d, idx, table)


def kernel(x, position_indices, pe_table):
    B, S, D = x.shape
    idx = position_indices.reshape(-1).astype(jnp.int32)
    out2d = _sc_gather_add(x.reshape(B * S, D), idx, pe_table)
    return out2d.reshape(B, S, D)
